# baseline (speedup 1.0000x reference)
import functools

import jax
import jax.numpy as jnp
from jax.experimental import pallas as pl
from jax.experimental.pallas import tpu as pltpu


_G = 4
_K = 4


def _relation_kernel(p_hbm, o_hbm, w_hbm, b_hbm, out_hbm,
                     p_v, o_v, w_v, b_v, r_v, sem, *,
                     f_num, n_p, n_o, d):
    n_pp, n_oo = f_num * n_p, f_num * n_o
    d_out = 2 * d // 2
    pg, og, wk = n_pp // _G, n_oo // _G, d // _K
    fg = f_num // _G

    idx = 0

    def start_copy(src, dst):
        nonlocal idx
        c = pltpu.make_async_copy(src, dst, sem.at[idx])
        c.start()
        idx += 1
        return c

    p_cp, o_cp, w_cp = [], [], []
    for g in range(_G):
        p_cp.append(start_copy(p_hbm.at[pl.ds(g * pg, pg)],
                               p_v.at[pl.ds(g * pg, pg)]))
        o_cp.append(start_copy(o_hbm.at[pl.ds(g * og, og)],
                               o_v.at[pl.ds(g * og, og)]))
        w_cp.append(start_copy(w_hbm.at[pl.ds(g * wk, wk)],
                               w_v.at[pl.ds(g * wk, wk)]))
    b_cp = start_copy(b_hbm, b_v)
    b_cp.wait()

    out_cp = []
    for g in range(_G):
        p_cp[g].wait()
        o_cp[g].wait()
        p_blk = p_v[pl.ds(g * pg, pg), :]
        o_blk = o_v[pl.ds(g * og, og), :]
        for k in range(_K):
            if g == 0:
                w_cp[k].wait()
            w_blk = w_v[pl.ds(k * wk, wk), :]
            wp = w_blk[:, :d]
            wo = w_blk[:, d:]
            a = jax.lax.dot_general(p_blk, wp, (((1,), (1,)), ((), ())),
                                    preferred_element_type=jnp.float32)
            bm = jax.lax.dot_general(o_blk, wo, (((1,), (1,)), ((), ())),
                                     preferred_element_type=jnp.float32)
            b_max = jnp.max(bm.reshape(fg, n_o, wk), axis=1)
            b_rep = jnp.broadcast_to(b_max[:, None, :], (fg, n_p, wk))
            r_v[pl.ds(g * pg, pg), pl.ds(k * wk, wk)] = (
                a + b_rep.reshape(pg, wk) + b_v[:, pl.ds(k * wk, wk)])
            out_cp.append(start_copy(
                r_v.at[pl.ds(g * pg, pg), pl.ds(k * wk, wk)],
                out_hbm.at[pl.ds(g * pg, pg), pl.ds(k * wk, wk)]))
    for c in out_cp:
        c.wait()


def kernel(person_features, other_features, person_boxes, other_boxes,
           is_person, W, b):
    f_num, n_p = person_boxes.shape[0], person_boxes.shape[1]
    n_o = other_boxes.shape[1]
    d = person_features.shape[1]
    d_out = W.shape[0]
    person = person_features.reshape(f_num * n_p, d)
    other = other_features.reshape(f_num * n_o, d)

    hbm = pltpu.MemorySpace.HBM
    out = pl.pallas_call(
        functools.partial(_relation_kernel, f_num=f_num, n_p=n_p, n_o=n_o, d=d),
        in_specs=[pl.BlockSpec(memory_space=hbm)] * 4,
        out_specs=pl.BlockSpec(memory_space=hbm),
        out_shape=jax.ShapeDtypeStruct((f_num * n_p, d_out), jnp.float32),
        scratch_shapes=[
            pltpu.VMEM((f_num * n_p, d), jnp.float32),
            pltpu.VMEM((f_num * n_o, d), jnp.float32),
            pltpu.VMEM((d_out, 2 * d), jnp.float32),
            pltpu.VMEM((1, d_out), jnp.float32),
            pltpu.VMEM((f_num * n_p, d_out), jnp.float32),
            pltpu.SemaphoreType.DMA((3 * _G + 1 + _G * _K,)),
        ],
    )(person, other, W, b.reshape(1, d_out))
    return out[:, :, None, None]


# speedup vs baseline: 1.1205x; 1.1205x over previous
import functools

import jax
import jax.numpy as jnp
from jax.experimental import pallas as pl
from jax.experimental.pallas import tpu as pltpu


_G = 4


def _relation_kernel(p_hbm, o_hbm, w_hbm, b_hbm, out_hbm,
                     p_v, o_v, w_v, b_v, r_v, sem, *,
                     f_num, n_p, n_o, d):
    n_pp, n_oo = f_num * n_p, f_num * n_o
    pg, og = n_pp // _G, n_oo // _G
    fg = f_num // _G

    idx = 0

    def start_copy(src, dst):
        nonlocal idx
        c = pltpu.make_async_copy(src, dst, sem.at[idx])
        c.start()
        idx += 1
        return c

    w_cp = [start_copy(w_hbm.at[pl.ds(r0, d // 2)], w_v.at[pl.ds(r0, d // 2)])
            for r0 in range(0, d, d // 2)]
    b_cp = start_copy(b_hbm, b_v)
    p_cp, o_cp = [], []
    for g in range(_G):
        p_cp.append(start_copy(p_hbm.at[pl.ds(g * pg, pg)],
                               p_v.at[pl.ds(g * pg, pg)]))
        o_cp.append(start_copy(o_hbm.at[pl.ds(g * og, og)],
                               o_v.at[pl.ds(g * og, og)]))
    for c in w_cp:
        c.wait()
    b_cp.wait()

    out_cp = []
    wp = w_v[:, :d]
    wo = w_v[:, d:]
    for g in range(_G):
        p_cp[g].wait()
        o_cp[g].wait()
        p_blk = p_v[pl.ds(g * pg, pg), :]
        o_blk = o_v[pl.ds(g * og, og), :]
        a = jax.lax.dot_general(p_blk, wp, (((1,), (1,)), ((), ())),
                                preferred_element_type=jnp.float32)
        bm = jax.lax.dot_general(o_blk, wo, (((1,), (1,)), ((), ())),
                                 preferred_element_type=jnp.float32)
        b_max = jnp.max(bm.reshape(fg, n_o, d), axis=1)
        b_rep = jnp.broadcast_to(b_max[:, None, :], (fg, n_p, d))
        r_v[pl.ds(g * pg, pg), :] = a + b_rep.reshape(pg, d) + b_v[:]
        out_cp.append(start_copy(r_v.at[pl.ds(g * pg, pg)],
                                 out_hbm.at[pl.ds(g * pg, pg)]))
    for c in out_cp:
        c.wait()


def kernel(person_features, other_features, person_boxes, other_boxes,
           is_person, W, b):
    f_num, n_p = person_boxes.shape[0], person_boxes.shape[1]
    n_o = other_boxes.shape[1]
    d = person_features.shape[1]
    d_out = W.shape[0]
    person = person_features.reshape(f_num * n_p, d)
    other = other_features.reshape(f_num * n_o, d)

    hbm = pltpu.MemorySpace.HBM
    out = pl.pallas_call(
        functools.partial(_relation_kernel, f_num=f_num, n_p=n_p, n_o=n_o, d=d),
        in_specs=[pl.BlockSpec(memory_space=hbm)] * 4,
        out_specs=pl.BlockSpec(memory_space=hbm),
        out_shape=jax.ShapeDtypeStruct((f_num * n_p, d_out), jnp.float32),
        scratch_shapes=[
            pltpu.VMEM((f_num * n_p, d), jnp.float32),
            pltpu.VMEM((f_num * n_o, d), jnp.float32),
            pltpu.VMEM((d_out, 2 * d), jnp.float32),
            pltpu.VMEM((1, d_out), jnp.float32),
            pltpu.VMEM((f_num * n_p, d_out), jnp.float32),
            pltpu.SemaphoreType.DMA((3 + 3 * _G,)),
        ],
    )(person, other, W, b.reshape(1, d_out))
    return out[:, :, None, None]


# speedup vs baseline: 1.1695x; 1.0437x over previous
import functools

import jax
import jax.numpy as jnp
from jax.experimental import pallas as pl


def _relation_kernel(person_ref, other_ref, w_ref, b_ref, out_ref, *,
                     frames, n_p, n_o, d):
    wp = w_ref[:, :d].astype(jnp.bfloat16)
    wo = w_ref[:, d:].astype(jnp.bfloat16)
    a = jax.lax.dot_general(person_ref[:].astype(jnp.bfloat16), wp,
                            (((1,), (1,)), ((), ())),
                            preferred_element_type=jnp.float32)
    b_mat = jax.lax.dot_general(other_ref[:].astype(jnp.bfloat16), wo,
                                (((1,), (1,)), ((), ())),
                                preferred_element_type=jnp.float32)
    b_max = jnp.max(b_mat.reshape(frames, n_o, d), axis=1)
    b_rep = jnp.broadcast_to(b_max[:, None, :], (frames, n_p, d))
    out_ref[:] = a + b_rep.reshape(frames * n_p, d) + b_ref[:]


def kernel(person_features, other_features, person_boxes, other_boxes,
           is_person, W, b):
    f_num, n_p = person_boxes.shape[0], person_boxes.shape[1]
    n_o = other_boxes.shape[1]
    d = person_features.shape[1]
    person = person_features.reshape(f_num * n_p, d)
    other = other_features.reshape(f_num * n_o, d)

    steps = 2 if f_num % 2 == 0 else 1
    frames_per_step = f_num // steps

    out = pl.pallas_call(
        functools.partial(_relation_kernel, frames=frames_per_step,
                          n_p=n_p, n_o=n_o, d=d),
        grid=(steps,),
        in_specs=[
            pl.BlockSpec((frames_per_step * n_p, d), lambda i: (i, 0)),
            pl.BlockSpec((frames_per_step * n_o, d), lambda i: (i, 0)),
            pl.BlockSpec((d, 2 * d), lambda i: (0, 0)),
            pl.BlockSpec((1, d), lambda i: (0, 0)),
        ],
        out_specs=pl.BlockSpec((frames_per_step * n_p, d), lambda i: (i, 0)),
        out_shape=jax.ShapeDtypeStruct((f_num * n_p, d), jnp.float32),
    )(person, other, W, b.reshape(1, d))
    return out[:, :, None, None]


# speedup vs baseline: 1.1750x; 1.0047x over previous
import functools

import jax
import jax.numpy as jnp
from jax.experimental import pallas as pl


def _relation_kernel(person_ref, other_ref, wp_ref, wo_ref, b_ref, out_ref, *,
                     frames, n_p, n_o, d):
    a = jax.lax.dot_general(person_ref[:], wp_ref[:], (((1,), (1,)), ((), ())),
                            preferred_element_type=jnp.float32)
    b_mat = jax.lax.dot_general(other_ref[:], wo_ref[:], (((1,), (1,)), ((), ())),
                                preferred_element_type=jnp.float32)
    b_max = jnp.max(b_mat.reshape(frames, n_o, d), axis=1)
    b_rep = jnp.broadcast_to(b_max[:, None, :], (frames, n_p, d))
    out_ref[:] = a + b_rep.reshape(frames * n_p, d) + b_ref[:]


def kernel(person_features, other_features, person_boxes, other_boxes,
           is_person, W, b):
    f_num, n_p = person_boxes.shape[0], person_boxes.shape[1]
    n_o = other_boxes.shape[1]
    d = person_features.shape[1]
    person = person_features.reshape(f_num * n_p, d)
    other = other_features.reshape(f_num * n_o, d)

    steps = 2 if f_num % 2 == 0 else 1
    frames_per_step = f_num // steps

    out = pl.pallas_call(
        functools.partial(_relation_kernel, frames=frames_per_step,
                          n_p=n_p, n_o=n_o, d=d),
        grid=(steps,),
        in_specs=[
            pl.BlockSpec((frames_per_step * n_p, d), lambda i: (i, 0)),
            pl.BlockSpec((frames_per_step * n_o, d), lambda i: (i, 0)),
            pl.BlockSpec((d, d), lambda i: (0, 0)),
            pl.BlockSpec((d, d), lambda i: (0, 1)),
            pl.BlockSpec((1, d), lambda i: (0, 0)),
        ],
        out_specs=pl.BlockSpec((frames_per_step * n_p, d), lambda i: (i, 0)),
        out_shape=jax.ShapeDtypeStruct((f_num * n_p, d), jnp.float32),
    )(person, other, W, W, b.reshape(1, d))
    return out[:, :, None, None]


# speedup vs baseline: 1.1804x; 1.0046x over previous
import functools

import jax
import jax.numpy as jnp
from jax.experimental import pallas as pl


def _relation_kernel(person_ref, other_ref, w_ref, b_ref, out_ref, *,
                     frames, n_p, n_o, d):
    wp = w_ref[:, :d]
    wo = w_ref[:, d:]
    a = jax.lax.dot_general(person_ref[:], wp, (((1,), (1,)), ((), ())),
                            preferred_element_type=jnp.float32)
    b_mat = jax.lax.dot_general(other_ref[:], wo, (((1,), (1,)), ((), ())),
                                preferred_element_type=jnp.float32)
    b_max = jnp.max(b_mat.reshape(frames, n_o, d), axis=1)
    b_rep = jnp.broadcast_to(b_max[:, None, :], (frames, n_p, d))
    out_ref[:] = a + b_rep.reshape(frames * n_p, d) + b_ref[:]


def kernel(person_features, other_features, person_boxes, other_boxes,
           is_person, W, b):
    f_num, n_p = person_boxes.shape[0], person_boxes.shape[1]
    n_o = other_boxes.shape[1]
    d = person_features.shape[1]
    person = person_features.reshape(f_num * n_p, d)
    other = other_features.reshape(f_num * n_o, d)

    steps = 2 if f_num % 2 == 0 else 1
    frames_per_step = f_num // steps

    out = pl.pallas_call(
        functools.partial(_relation_kernel, frames=frames_per_step,
                          n_p=n_p, n_o=n_o, d=d),
        grid=(steps,),
        in_specs=[
            pl.BlockSpec((frames_per_step * n_p, d), lambda i: (i, 0)),
            pl.BlockSpec((frames_per_step * n_o, d), lambda i: (i, 0)),
            pl.BlockSpec((d, 2 * d), lambda i: (0, 0)),
            pl.BlockSpec((1, d), lambda i: (0, 0)),
        ],
        out_specs=pl.BlockSpec((frames_per_step * n_p, d), lambda i: (i, 0)),
        out_shape=jax.ShapeDtypeStruct((f_num * n_p, d), jnp.float32),
    )(person, other, W, b.reshape(1, d))
    return out[:, :, None, None]
